# Initial kernel scaffold; baseline (speedup 1.0000x reference)
import jax
import jax.numpy as jnp
from jax.experimental import pallas as pl


def kernel(delta_t, pe):
    raise NotImplementedError("write your pallas kernel here")


# speedup vs baseline: 5.4809x; 5.4809x over previous
import functools

import jax
import jax.numpy as jnp
from jax import lax
from jax.experimental import pallas as pl
from jax.experimental.pallas import tpu as pltpu
from jax.experimental.pallas import tpu_sc as plsc

D_MODEL = 128
BATCH = 1024
T = 200
B_TOTAL = BATCH * T
NC, NS = 2, 16
NW = NC * NS
B_PER_W = B_TOTAL // NW
CHUNK = 128
N_CHUNKS = B_PER_W // CHUNK


def _make_gather():
    mesh = plsc.VectorSubcoreMesh(core_axis_name="c", subcore_axis_name="s")

    @functools.partial(
        pl.kernel,
        mesh=mesh,
        out_type=jax.ShapeDtypeStruct((B_TOTAL, D_MODEL), jnp.float32),
        scratch_types=[
            pltpu.VMEM((B_PER_W,), jnp.int32),
            pltpu.VMEM((CHUNK, D_MODEL), jnp.float32),
            pltpu.SemaphoreType.DMA,
        ],
    )
    def gather_kernel(idx_hbm, table_hbm, out_hbm, idx_v, rows_v, gsem):
        wid = lax.axis_index("s") * NC + lax.axis_index("c")
        base = wid * B_PER_W
        pltpu.sync_copy(idx_hbm.at[pl.ds(base, B_PER_W)], idx_v)

        def chunk_body(c, carry):
            off = c * CHUNK
            pltpu.async_copy(
                table_hbm.at[idx_v.at[pl.ds(off, CHUNK)]], rows_v, gsem
            ).wait()
            pltpu.sync_copy(rows_v, out_hbm.at[pl.ds(base + off, CHUNK)])
            return carry

        lax.fori_loop(0, N_CHUNKS, chunk_body, 0)

    return gather_kernel


_gather = _make_gather()


def kernel(delta_t, pe):
    idx = delta_t.reshape(-1)
    table = pe[0]
    out = _gather(idx, table)
    return out.reshape(BATCH, T, D_MODEL)


# speedup vs baseline: 7.2420x; 1.3213x over previous
import functools

import jax
import jax.numpy as jnp
from jax import lax
from jax.experimental import pallas as pl
from jax.experimental.pallas import tpu as pltpu
from jax.experimental.pallas import tpu_sc as plsc

D_MODEL = 128
BATCH = 1024
T = 200
B_TOTAL = BATCH * T
NC, NS = 2, 16
NW = NC * NS
B_PER_W = B_TOTAL // NW
CHUNK = 128
N_CHUNKS = B_PER_W // CHUNK


def _make_gather():
    mesh = plsc.VectorSubcoreMesh(core_axis_name="c", subcore_axis_name="s")

    @functools.partial(
        pl.kernel,
        mesh=mesh,
        out_type=jax.ShapeDtypeStruct((B_TOTAL, D_MODEL), jnp.float32),
        scratch_types=[
            pltpu.VMEM((B_PER_W,), jnp.int32),
            pltpu.VMEM((CHUNK, D_MODEL), jnp.float32),
            pltpu.VMEM((CHUNK, D_MODEL), jnp.float32),
            pltpu.SemaphoreType.DMA,
            pltpu.SemaphoreType.DMA,
        ],
    )
    def gather_kernel(idx_hbm, table_hbm, out_hbm, idx_v, rows_a, rows_b, sem_a, sem_b):
        wid = lax.axis_index("s") * NC + lax.axis_index("c")
        base = wid * B_PER_W
        pltpu.sync_copy(idx_hbm.at[pl.ds(base, B_PER_W)], idx_v)

        bufs = (rows_a, rows_b)
        sems = (sem_a, sem_b)

        def fire(c, b):
            pltpu.async_copy(
                table_hbm.at[idx_v.at[pl.ds(c * CHUNK, CHUNK)]], bufs[b], sems[b]
            )

        def drain(b):
            pltpu.make_async_copy(
                table_hbm.at[pl.ds(0, CHUNK)], bufs[b], sems[b]
            ).wait()

        fire(0, 0)
        fire(1, 1)

        def chunk_pair(j, carry):
            for b in range(2):
                c = 2 * j + b
                drain(b)
                pltpu.sync_copy(bufs[b], out_hbm.at[pl.ds(base + c * CHUNK, CHUNK)])
                fire(c + 2, b)
            return carry

        lax.fori_loop(0, (N_CHUNKS - 2) // 2, chunk_pair, 0)

        for b in range(2):
            c = N_CHUNKS - 2 + b
            drain(b)
            pltpu.sync_copy(bufs[b], out_hbm.at[pl.ds(base + c * CHUNK, CHUNK)])

    return gather_kernel


_gather = _make_gather()


def kernel(delta_t, pe):
    idx = delta_t.reshape(-1)
    table = pe[0]
    out = _gather(idx, table)
    return out.reshape(BATCH, T, D_MODEL)


# speedup vs baseline: 7.2820x; 1.0055x over previous
import functools

import jax
import jax.numpy as jnp
from jax import lax
from jax.experimental import pallas as pl
from jax.experimental.pallas import tpu as pltpu
from jax.experimental.pallas import tpu_sc as plsc

D_MODEL = 128
BATCH = 1024
T = 200
B_TOTAL = BATCH * T
NC, NS = 2, 16
NW = NC * NS
B_PER_W = B_TOTAL // NW
CHUNK = 128
N_CHUNKS = B_PER_W // CHUNK


def _make_gather():
    mesh = plsc.VectorSubcoreMesh(core_axis_name="c", subcore_axis_name="s")

    nbuf = 4

    @functools.partial(
        pl.kernel,
        mesh=mesh,
        out_type=jax.ShapeDtypeStruct((B_TOTAL, D_MODEL), jnp.float32),
        scratch_types=[
            pltpu.VMEM((B_PER_W,), jnp.int32),
        ]
        + [pltpu.VMEM((CHUNK, D_MODEL), jnp.float32)] * nbuf
        + [pltpu.SemaphoreType.DMA] * (2 * nbuf),
    )
    def gather_kernel(idx_hbm, table_hbm, out_hbm, idx_v, *scratch):
        bufs = scratch[:nbuf]
        gsems = scratch[nbuf : 2 * nbuf]
        ssems = scratch[2 * nbuf :]

        wid = lax.axis_index("s") * NC + lax.axis_index("c")
        base = wid * B_PER_W
        pltpu.sync_copy(idx_hbm.at[pl.ds(base, B_PER_W)], idx_v)

        def fire_g(c, b):
            pltpu.async_copy(
                table_hbm.at[idx_v.at[pl.ds(c * CHUNK, CHUNK)]], bufs[b], gsems[b]
            )

        def wait_g(b):
            pltpu.make_async_copy(
                table_hbm.at[pl.ds(0, CHUNK)], bufs[b], gsems[b]
            ).wait()

        def fire_s(c, b):
            pltpu.async_copy(
                bufs[b], out_hbm.at[pl.ds(base + c * CHUNK, CHUNK)], ssems[b]
            )

        def wait_s(b):
            pltpu.make_async_copy(
                bufs[b], out_hbm.at[pl.ds(base, CHUNK)], ssems[b]
            ).wait()

        fire_g(0, 0)
        fire_g(1, 1)
        for c in (0, 1):
            b = c % nbuf
            wait_g(b)
            fire_s(c, b)
            fire_g(c + 2, (c + 2) % nbuf)

        def ring(j, carry):
            for i in range(4):
                c = 2 + 4 * j + i
                b = (2 + i) % nbuf
                wait_g(b)
                fire_s(c, b)
                wait_s((b + 2) % nbuf)
                fire_g(c + 2, (b + 2) % nbuf)
            return carry

        n_uniform = N_CHUNKS - 4
        lax.fori_loop(0, n_uniform // 4, ring, 0)
        for c in range(2 + (n_uniform // 4) * 4, N_CHUNKS - 2):
            b = c % nbuf
            wait_g(b)
            fire_s(c, b)
            wait_s((b + 2) % nbuf)
            fire_g(c + 2, (b + 2) % nbuf)
        for c in (N_CHUNKS - 2, N_CHUNKS - 1):
            b = c % nbuf
            wait_g(b)
            fire_s(c, b)
            wait_s((b + 2) % nbuf)
        wait_s((N_CHUNKS - 2) % nbuf)
        wait_s((N_CHUNKS - 1) % nbuf)

    return gather_kernel


_gather = _make_gather()


def kernel(delta_t, pe):
    idx = delta_t.reshape(-1)
    table = pe[0]
    out = _gather(idx, table)
    return out.reshape(BATCH, T, D_MODEL)
